# Initial kernel scaffold; baseline (speedup 1.0000x reference)
import jax
import jax.numpy as jnp
from jax.experimental import pallas as pl


def kernel(input_seq, embedding_table):
    raise NotImplementedError("write your pallas kernel here")


# speedup vs baseline: 1.0942x; 1.0942x over previous
import functools

import jax
import jax.numpy as jnp
from jax import lax
from jax.experimental import pallas as pl
from jax.experimental.pallas import tpu as pltpu
from jax.experimental.pallas import tpu_sc as plsc

NUM_CORES = 2
NUM_SUBCORES = 16
NUM_WORKERS = NUM_CORES * NUM_SUBCORES

CHUNK = 1024


@functools.lru_cache(maxsize=None)
def _make_lookup(V, D, B):
    assert B % (NUM_WORKERS * CHUNK) == 0
    b_per_w = B // NUM_WORKERS
    n_chunks = b_per_w // CHUNK
    mesh = plsc.VectorSubcoreMesh(core_axis_name="c", subcore_axis_name="s")

    @functools.partial(
        pl.kernel,
        mesh=mesh,
        out_type=jax.ShapeDtypeStruct((B, D), jnp.float32),
        scratch_types=[
            pltpu.VMEM((CHUNK,), jnp.int32),
            pltpu.VMEM((CHUNK, D), jnp.float32),
            pltpu.SemaphoreType.DMA,
        ],
        compiler_params=pltpu.CompilerParams(use_tc_tiling_on_sc=False),
    )
    def lookup(table_hbm, idx_hbm, out_hbm, idx_v, rows_v, sem):
        wid = lax.axis_index("s") * NUM_CORES + lax.axis_index("c")
        base = wid * b_per_w

        def body(c, carry):
            off = base + c * CHUNK
            pltpu.sync_copy(idx_hbm.at[pl.ds(off, CHUNK)], idx_v)
            pltpu.async_copy(table_hbm.at[idx_v], rows_v, sem).wait()
            pltpu.sync_copy(rows_v, out_hbm.at[pl.ds(off, CHUNK)])
            return carry

        lax.fori_loop(0, n_chunks, body, 0)

    return lookup


def kernel(input_seq, embedding_table):
    Bt, H = input_seq.shape
    V, D = embedding_table.shape
    B = Bt * H
    idx = input_seq.reshape(B).astype(jnp.int32)
    out = _make_lookup(V, D, B)(embedding_table, idx)
    return out.reshape(Bt, H, D)


# speedup vs baseline: 1.1139x; 1.0180x over previous
import functools

import jax
import jax.numpy as jnp
from jax import lax
from jax.experimental import pallas as pl
from jax.experimental.pallas import tpu as pltpu
from jax.experimental.pallas import tpu_sc as plsc

NUM_CORES = 2
NUM_SUBCORES = 16
NUM_WORKERS = NUM_CORES * NUM_SUBCORES

CHUNK = 640
D_BUF = 4


@functools.lru_cache(maxsize=None)
def _make_lookup(V, D, B):
    assert B % (NUM_WORKERS * CHUNK * D_BUF) == 0
    b_per_w = B // NUM_WORKERS
    n_chunks = b_per_w // CHUNK
    n_super = n_chunks // D_BUF
    mesh = plsc.VectorSubcoreMesh(core_axis_name="c", subcore_axis_name="s")

    @functools.partial(
        pl.kernel,
        mesh=mesh,
        out_type=jax.ShapeDtypeStruct((B, D), jnp.float32),
        scratch_types=[
            pltpu.VMEM((b_per_w,), jnp.int32),
            pltpu.VMEM((D_BUF, CHUNK, D), jnp.float32),
            pltpu.SemaphoreType.DMA((D_BUF,)),
            pltpu.SemaphoreType.DMA((D_BUF,)),
        ],
        compiler_params=pltpu.CompilerParams(use_tc_tiling_on_sc=False),
    )
    def lookup(table_hbm, idx_hbm, out_hbm, idx_v, rows_v, sem_g, sem_o):
        wid = lax.axis_index("s") * NUM_CORES + lax.axis_index("c")
        base = wid * b_per_w
        pltpu.sync_copy(idx_hbm.at[pl.ds(base, b_per_w)], idx_v)

        def gather_start(g, b):
            idx_slice = idx_v.at[pl.ds(g * CHUNK, CHUNK)]
            pltpu.async_copy(table_hbm.at[idx_slice], rows_v.at[b], sem_g.at[b])

        def gather_wait(b):
            pltpu.make_async_copy(
                table_hbm.at[idx_v.at[pl.ds(0, CHUNK)]], rows_v.at[b], sem_g.at[b]
            ).wait()

        def write_start(c, b):
            pltpu.async_copy(
                rows_v.at[b], out_hbm.at[pl.ds(base + c * CHUNK, CHUNK)], sem_o.at[b]
            )

        def write_wait(b):
            pltpu.make_async_copy(
                rows_v.at[b], out_hbm.at[pl.ds(0, CHUNK)], sem_o.at[b]
            ).wait()

        gather_start(0, 0)
        gather_start(1, 1)

        def super_body(s, carry):
            for j in range(D_BUF):
                c = s * D_BUF + j
                bg = (j + 2) % D_BUF
                if j < 2:
                    @pl.when(s > 0)
                    def _():
                        write_wait(bg)

                    gather_start(c + 2, bg)
                else:
                    @pl.when(s < n_super - 1)
                    def _():
                        write_wait(bg)
                        gather_start(c + 2, bg)

                gather_wait(j)
                write_start(c, j)
            return carry

        lax.fori_loop(0, n_super, super_body, 0)

        for b in range(D_BUF):
            write_wait(b)

    return lookup


def kernel(input_seq, embedding_table):
    Bt, H = input_seq.shape
    V, D = embedding_table.shape
    B = Bt * H
    idx = input_seq.reshape(B).astype(jnp.int32)
    out = _make_lookup(V, D, B)(embedding_table, idx)
    return out.reshape(Bt, H, D)


# speedup vs baseline: 1.7694x; 1.5885x over previous
import functools

import jax
import jax.numpy as jnp
from jax import lax
from jax.experimental import pallas as pl
from jax.experimental.pallas import tpu as pltpu
from jax.experimental.pallas import tpu_sc as plsc

NUM_CORES = 2
NUM_SUBCORES = 16
NUM_WORKERS = NUM_CORES * NUM_SUBCORES

CH_B = 8
D_BUF = 4
LANES = 128
SUBLANE = 8


@functools.lru_cache(maxsize=None)
def _make_lookup(V, D, Bt, H):
    Hp = (H + SUBLANE - 1) // SUBLANE * SUBLANE
    rows_per_w = Bt // NUM_WORKERS
    n_chunks = rows_per_w // CH_B
    n_super = n_chunks // D_BUF
    idx_per_chunk = CH_B * H
    b_per_w = rows_per_w * H
    assert rows_per_w % CH_B == 0 and n_chunks % D_BUF == 0
    mesh = plsc.VectorSubcoreMesh(core_axis_name="c", subcore_axis_name="s")

    @functools.partial(
        pl.kernel,
        mesh=mesh,
        out_type=jax.ShapeDtypeStruct((Bt * Hp, LANES), jnp.float32),
        scratch_types=[
            pltpu.VMEM((b_per_w,), jnp.int32),
            pltpu.VMEM((D_BUF, idx_per_chunk, D), jnp.float32),
            pltpu.SemaphoreType.DMA((D_BUF,)),
            pltpu.SemaphoreType.DMA((D_BUF,)),
        ],
        compiler_params=pltpu.CompilerParams(use_tc_tiling_on_sc=False),
    )
    def lookup(tab_hbm, idx_hbm, out_hbm, idx_v, rows_v, sem_g, sem_o):
        wid = lax.axis_index("s") * NUM_CORES + lax.axis_index("c")
        row0 = wid * rows_per_w
        pltpu.sync_copy(idx_hbm.at[pl.ds(row0 * H, b_per_w)], idx_v)

        def gather_start(g, b):
            idx_slice = idx_v.at[pl.ds(g * idx_per_chunk, idx_per_chunk)]
            pltpu.async_copy(tab_hbm.at[idx_slice], rows_v.at[b], sem_g.at[b])

        def gather_wait(b):
            pltpu.make_async_copy(
                tab_hbm.at[idx_v.at[pl.ds(0, idx_per_chunk)]],
                rows_v.at[b],
                sem_g.at[b],
            ).wait()

        def write_start(c, b):
            for j in range(CH_B):
                brow = row0 + c * CH_B + j
                pltpu.async_copy(
                    rows_v.at[b, pl.ds(j * H, H), :],
                    out_hbm.at[pl.ds(brow * Hp, H), pl.ds(0, D)],
                    sem_o.at[b],
                )

        def write_wait(b):
            for _ in range(CH_B):
                pltpu.make_async_copy(
                    rows_v.at[b, pl.ds(0, H), :],
                    out_hbm.at[pl.ds(0, H), pl.ds(0, D)],
                    sem_o.at[b],
                ).wait()

        gather_start(0, 0)
        gather_start(1, 1)

        def super_body(s, carry):
            for j in range(D_BUF):
                c = s * D_BUF + j
                bg = (j + 2) % D_BUF
                if j < 2:
                    @pl.when(s > 0)
                    def _():
                        write_wait(bg)

                    gather_start(c + 2, bg)
                else:
                    @pl.when(s < n_super - 1)
                    def _():
                        write_wait(bg)
                        gather_start(c + 2, bg)

                gather_wait(j)
                write_start(c, j)
            return carry

        lax.fori_loop(0, n_super, super_body, 0)

        for b in range(D_BUF):
            write_wait(b)

    return lookup


def kernel(input_seq, embedding_table):
    Bt, H = input_seq.shape
    V, D = embedding_table.shape
    Hp = (H + SUBLANE - 1) // SUBLANE * SUBLANE
    idx = jnp.minimum(input_seq.reshape(Bt * H).astype(jnp.int32),
                      jnp.int32(V - 1))
    tab = jnp.minimum(embedding_table, jnp.float32(3.4028235e38))
    out2 = _make_lookup(V, D, Bt, H)(tab, idx)
    out3 = out2.reshape(Bt, Hp, LANES)
    return out3[:, :H, :D]
